# Initial kernel scaffold; baseline (speedup 1.0000x reference)
import functools

import jax
import jax.numpy as jnp
from jax import lax
from jax.experimental import pallas as pl
from jax.experimental.pallas import tpu as pltpu
from jax.experimental.pallas import tpu_sc as plsc


def _mm_body(x_ref, w_ref, o_ref):
    o_ref[0] = jnp.dot(x_ref[...], w_ref[0], preferred_element_type=jnp.float32)


def _rel_matmul(x, w, block_n):
    n, d_in = x.shape
    r, _, d_out = w.shape
    grid = (r, n // block_n)
    return pl.pallas_call(
        _mm_body,
        grid=grid,
        in_specs=[
            pl.BlockSpec((block_n, d_in), lambda ri, i: (i, 0)),
            pl.BlockSpec((1, d_in, d_out), lambda ri, i: (ri, 0, 0)),
        ],
        out_specs=pl.BlockSpec((1, block_n, d_out), lambda ri, i: (ri, i, 0)),
        out_shape=jax.ShapeDtypeStruct((r, n, d_out), jnp.float32),
    )(x, w)


def _relu_mm_body(p_ref, w_ref, o_ref):
    h = jnp.maximum(p_ref[0] + p_ref[1], 0.0)
    o_ref[0] = jnp.dot(h, w_ref[0], preferred_element_type=jnp.float32)


def _relu_sum_matmul(p, w, block_n):
    _, n, d_h = p.shape
    r, _, d_out = w.shape
    grid = (r, n // block_n)
    return pl.pallas_call(
        _relu_mm_body,
        grid=grid,
        in_specs=[
            pl.BlockSpec((2, block_n, d_h), lambda ri, i: (0, i, 0)),
            pl.BlockSpec((1, d_h, d_out), lambda ri, i: (ri, 0, 0)),
        ],
        out_specs=pl.BlockSpec((1, block_n, d_out), lambda ri, i: (ri, i, 0)),
        out_shape=jax.ShapeDtypeStruct((r, n, d_out), jnp.float32),
    )(p, w)


def _pair_sum_body(p_ref, o_ref):
    o_ref[...] = p_ref[0] + p_ref[1]


def _pair_sum(p, block_n):
    _, n, d = p.shape
    return pl.pallas_call(
        _pair_sum_body,
        grid=(n // block_n,),
        in_specs=[pl.BlockSpec((2, block_n, d), lambda i: (0, i, 0))],
        out_specs=pl.BlockSpec((block_n, d), lambda i: (i, 0)),
        out_shape=jax.ShapeDtypeStruct((n, d), jnp.float32),
    )(p)


_CHUNK = 80


def _sc_gather_scatter(table, src, dst, etype, n_nodes, d):
    info = plsc.get_sparse_core_info()
    nc, ns = info.num_cores, info.num_subcores
    nw = nc * ns
    e = src.shape[0]
    assert e % (nw * _CHUNK) == 0
    epw = e // nw
    nchunk = epw // _CHUNK
    assert n_nodes % ns == 0
    rpt = n_nodes // ns
    zr = 125
    assert rpt % zr == 0
    nz = rpt // zr

    @functools.partial(
        pl.kernel,
        out_type=jax.ShapeDtypeStruct((nc, n_nodes, d), jnp.float32),
        mesh=plsc.VectorSubcoreMesh(core_axis_name="c", subcore_axis_name="s"),
        scratch_types=[
            pltpu.VMEM((_CHUNK,), jnp.int32),
            pltpu.VMEM((_CHUNK,), jnp.int32),
            pltpu.VMEM((_CHUNK,), jnp.int32),
            pltpu.VMEM((_CHUNK,), jnp.int32),
            pltpu.VMEM((_CHUNK, d), jnp.float32),
            pltpu.VMEM((zr, d), jnp.float32),
            pltpu.VMEM_SHARED((n_nodes, d), jnp.float32),
            pltpu.SemaphoreType.DMA,
        ],
    )
    def k(table_h, src_h, dst_h, et_h, out_h,
          srcv, etv, gidx, dstv, rows, stage, acc, sem):
        c = lax.axis_index("c")
        s = lax.axis_index("s")
        wid = s * nc + c

        def zrow(i, carry):
            for jj in range(d // 16):
                stage[i, pl.ds(jj * 16, 16)] = jnp.zeros((16,), jnp.float32)
            return carry

        lax.fori_loop(0, zr, zrow, 0)
        r0 = s * rpt
        for kk in range(nz):
            pltpu.sync_copy(stage, acc.at[pl.ds(r0 + kk * zr, zr)])
        plsc.subcore_barrier()

        ebase = wid * epw

        def body(j, carry):
            b = ebase + j * _CHUNK
            pltpu.sync_copy(src_h.at[pl.ds(b, _CHUNK)], srcv)
            pltpu.sync_copy(et_h.at[pl.ds(b, _CHUNK)], etv)
            pltpu.sync_copy(dst_h.at[pl.ds(b, _CHUNK)], dstv)
            for i in range(_CHUNK // 16):
                sl = pl.ds(i * 16, 16)
                gidx[sl] = etv[sl] * n_nodes + srcv[sl]
            pltpu.async_copy(table_h.at[gidx], rows, sem).wait()
            pltpu.sync_copy(rows, acc.at[dstv], add=True)
            return carry

        lax.fori_loop(0, nchunk, body, 0)
        plsc.subcore_barrier()

        for kk in range(nz):
            pltpu.sync_copy(acc.at[pl.ds(r0 + kk * zr, zr)], stage)
            pltpu.sync_copy(stage, out_h.at[c].at[pl.ds(r0 + kk * zr, zr)])

    return k(table, src, dst, etype)


def kernel(x, edge_index, edge_type, W1, W2):
    src = edge_index[0]
    dst = edge_index[1]
    n, _ = x.shape
    r, _, d_h = W1.shape
    d_out = W2.shape[2]

    t1 = _rel_matmul(x, W1, block_n=2000).reshape(r * n, d_h)
    p1 = _sc_gather_scatter(t1, src, dst, edge_type, n, d_h)
    t2 = _relu_sum_matmul(p1, W2, block_n=2000).reshape(r * n, d_out)
    p2 = _sc_gather_scatter(t2, src, dst, edge_type, n, d_out)
    return _pair_sum(p2, block_n=2000)


# speedup vs baseline: 18.2992x; 18.2992x over previous
import functools

import jax
import jax.numpy as jnp
from jax import lax
from jax.experimental import pallas as pl
from jax.experimental.pallas import tpu as pltpu
from jax.experimental.pallas import tpu_sc as plsc


def _mm_body(x_ref, w_ref, o_ref):
    o_ref[0] = jnp.dot(x_ref[...], w_ref[0], preferred_element_type=jnp.float32)


def _rel_matmul(x, w, block_n):
    n, d_in = x.shape
    r, _, d_out = w.shape
    grid = (r, n // block_n)
    return pl.pallas_call(
        _mm_body,
        grid=grid,
        in_specs=[
            pl.BlockSpec((block_n, d_in), lambda ri, i: (i, 0)),
            pl.BlockSpec((1, d_in, d_out), lambda ri, i: (ri, 0, 0)),
        ],
        out_specs=pl.BlockSpec((1, block_n, d_out), lambda ri, i: (ri, i, 0)),
        out_shape=jax.ShapeDtypeStruct((r, n, d_out), jnp.float32),
    )(x, w)


def _relu_mm_body(p_ref, w_ref, o_ref):
    h = jnp.maximum(p_ref[0] + p_ref[1], 0.0)
    o_ref[0] = jnp.dot(h, w_ref[0], preferred_element_type=jnp.float32)


def _relu_sum_matmul(p, w, block_n):
    _, n, d_h = p.shape
    r, _, d_out = w.shape
    grid = (r, n // block_n)
    return pl.pallas_call(
        _relu_mm_body,
        grid=grid,
        in_specs=[
            pl.BlockSpec((2, block_n, d_h), lambda ri, i: (0, i, 0)),
            pl.BlockSpec((1, d_h, d_out), lambda ri, i: (ri, 0, 0)),
        ],
        out_specs=pl.BlockSpec((1, block_n, d_out), lambda ri, i: (ri, i, 0)),
        out_shape=jax.ShapeDtypeStruct((r, n, d_out), jnp.float32),
    )(p, w)


def _pair_sum_body(p_ref, o_ref):
    o_ref[...] = p_ref[0] + p_ref[1]


def _pair_sum(p, block_n):
    _, n, d = p.shape
    return pl.pallas_call(
        _pair_sum_body,
        grid=(n // block_n,),
        in_specs=[pl.BlockSpec((2, block_n, d), lambda i: (0, i, 0))],
        out_specs=pl.BlockSpec((block_n, d), lambda i: (i, 0)),
        out_shape=jax.ShapeDtypeStruct((n, d), jnp.float32),
    )(p)


_CHUNK = 80


def _sc_gather_scatter(table, src, dst, etype, n_nodes, d):
    info = plsc.get_sparse_core_info()
    nc, ns = info.num_cores, info.num_subcores
    nw = nc * ns
    e = src.shape[0]
    assert e % (nw * _CHUNK) == 0
    epw = e // nw
    nchunk = epw // _CHUNK
    rpt = (n_nodes // ns) // 8 * 8
    rem = n_nodes - rpt * ns
    assert rem % 8 == 0 and rem <= rpt

    @functools.partial(
        pl.kernel,
        out_type=jax.ShapeDtypeStruct((nc, n_nodes, d), jnp.float32),
        mesh=plsc.VectorSubcoreMesh(core_axis_name="c", subcore_axis_name="s"),
        scratch_types=[
            pltpu.VMEM((_CHUNK,), jnp.int32),
            pltpu.VMEM((_CHUNK,), jnp.int32),
            pltpu.VMEM((_CHUNK,), jnp.int32),
            pltpu.VMEM((_CHUNK,), jnp.int32),
            pltpu.VMEM((_CHUNK, d), jnp.float32),
            pltpu.VMEM((rpt, d), jnp.float32),
            pltpu.VMEM_SHARED((n_nodes, d), jnp.float32),
            pltpu.SemaphoreType.DMA,
        ],
        compiler_params=pltpu.CompilerParams(use_tc_tiling_on_sc=False),
    )
    def k(table_h, src_h, dst_h, et_h, out_h,
          srcv, etv, gidx, dstv, rows, stage, acc, sem):
        c = lax.axis_index("c")
        s = lax.axis_index("s")
        wid = s * nc + c

        def zrow(i, carry):
            for jj in range(d // 16):
                stage[i, pl.ds(jj * 16, 16)] = jnp.zeros((16,), jnp.float32)
            return carry

        lax.fori_loop(0, rpt, zrow, 0)
        r0 = pl.multiple_of(s * rpt, 8)
        pltpu.sync_copy(stage, acc.at[pl.ds(r0, rpt)])

        @pl.when(s == 0)
        def _():
            pltpu.sync_copy(stage.at[pl.ds(0, rem)],
                            acc.at[pl.ds(ns * rpt, rem)])

        plsc.subcore_barrier()

        ebase = wid * epw

        def body(j, carry):
            b = ebase + j * _CHUNK
            pltpu.sync_copy(src_h.at[pl.ds(b, _CHUNK)], srcv)
            pltpu.sync_copy(et_h.at[pl.ds(b, _CHUNK)], etv)
            pltpu.sync_copy(dst_h.at[pl.ds(b, _CHUNK)], dstv)
            for i in range(_CHUNK // 16):
                sl = pl.ds(i * 16, 16)
                gidx[sl] = etv[sl] * n_nodes + srcv[sl]
            pltpu.async_copy(table_h.at[gidx], rows, sem).wait()
            pltpu.sync_copy(rows, acc.at[dstv], add=True)
            return carry

        lax.fori_loop(0, nchunk, body, 0)
        plsc.subcore_barrier()

        pltpu.sync_copy(acc.at[pl.ds(r0, rpt)], stage)
        pltpu.sync_copy(stage, out_h.at[c].at[pl.ds(r0, rpt)])

        @pl.when(s == 0)
        def _():
            pltpu.sync_copy(acc.at[pl.ds(ns * rpt, rem)],
                            stage.at[pl.ds(0, rem)])
            pltpu.sync_copy(stage.at[pl.ds(0, rem)],
                            out_h.at[c].at[pl.ds(ns * rpt, rem)])

    return k(table, src, dst, etype)


def kernel(x, edge_index, edge_type, W1, W2):
    src = edge_index[0]
    dst = edge_index[1]
    n, _ = x.shape
    r, _, d_h = W1.shape
    d_out = W2.shape[2]

    t1 = _rel_matmul(x, W1, block_n=2000).reshape(r * n, d_h)
    p1 = _sc_gather_scatter(t1, src, dst, edge_type, n, d_h)
    t2 = _relu_sum_matmul(p1, W2, block_n=2000).reshape(r * n, d_out)
    p2 = _sc_gather_scatter(t2, src, dst, edge_type, n, d_out)
    return _pair_sum(p2, block_n=2000)


# speedup vs baseline: 61.5116x; 3.3614x over previous
import functools

import jax
import jax.numpy as jnp
from jax import lax
from jax.experimental import pallas as pl
from jax.experimental.pallas import tpu as pltpu
from jax.experimental.pallas import tpu_sc as plsc


def _mm_body(x_ref, w_ref, o_ref):
    o_ref[0] = jnp.dot(x_ref[...], w_ref[0], preferred_element_type=jnp.float32)


def _rel_matmul(x, w, block_n):
    n, d_in = x.shape
    r, _, d_out = w.shape
    grid = (r, n // block_n)
    return pl.pallas_call(
        _mm_body,
        grid=grid,
        in_specs=[
            pl.BlockSpec((block_n, d_in), lambda ri, i: (i, 0)),
            pl.BlockSpec((1, d_in, d_out), lambda ri, i: (ri, 0, 0)),
        ],
        out_specs=pl.BlockSpec((1, block_n, d_out), lambda ri, i: (ri, i, 0)),
        out_shape=jax.ShapeDtypeStruct((r, n, d_out), jnp.float32),
    )(x, w)


def _relu_mm_body(p_ref, w_ref, o_ref):
    h = jnp.maximum(p_ref[0] + p_ref[1], 0.0)
    o_ref[0] = jnp.dot(h, w_ref[0], preferred_element_type=jnp.float32)


def _relu_sum_matmul(p, w, block_n):
    _, n, d_h = p.shape
    r, _, d_out = w.shape
    grid = (r, n // block_n)
    return pl.pallas_call(
        _relu_mm_body,
        grid=grid,
        in_specs=[
            pl.BlockSpec((2, block_n, d_h), lambda ri, i: (0, i, 0)),
            pl.BlockSpec((1, d_h, d_out), lambda ri, i: (ri, 0, 0)),
        ],
        out_specs=pl.BlockSpec((1, block_n, d_out), lambda ri, i: (ri, i, 0)),
        out_shape=jax.ShapeDtypeStruct((r, n, d_out), jnp.float32),
    )(p, w)


def _pair_sum_body(p_ref, o_ref):
    o_ref[...] = p_ref[0] + p_ref[1]


def _pair_sum(p, block_n):
    _, n, d = p.shape
    return pl.pallas_call(
        _pair_sum_body,
        grid=(n // block_n,),
        in_specs=[pl.BlockSpec((2, block_n, d), lambda i: (0, i, 0))],
        out_specs=pl.BlockSpec((block_n, d), lambda i: (i, 0)),
        out_shape=jax.ShapeDtypeStruct((n, d), jnp.float32),
    )(p)


_CHUNK = 80
_NBUF = 5


def _sc_gather_scatter(table, src3, dst3, et3, n_nodes, d):
    info = plsc.get_sparse_core_info()
    nc, ns = info.num_cores, info.num_subcores
    nw = nc * ns
    nchunk = src3.shape[1]
    epw = nchunk * _CHUNK
    assert nchunk % _NBUF == 0
    ngroups = nchunk // _NBUF
    rpt = (n_nodes // ns) // 8 * 8
    rem = n_nodes - rpt * ns
    assert rem % 8 == 0 and rem <= rpt
    zr = 208
    assert rpt % zr == 0
    nz = rpt // zr

    @functools.partial(
        pl.kernel,
        out_type=jax.ShapeDtypeStruct((nc, n_nodes, d), jnp.float32),
        mesh=plsc.VectorSubcoreMesh(core_axis_name="c", subcore_axis_name="s"),
        scratch_types=[
            pltpu.VMEM((nchunk, _CHUNK), jnp.int32),
            pltpu.VMEM((nchunk, _CHUNK), jnp.int32),
            pltpu.VMEM((nchunk, _CHUNK), jnp.int32),
            [pltpu.VMEM((_CHUNK, d), jnp.float32) for _ in range(_NBUF)],
            pltpu.VMEM((zr, d), jnp.float32),
            pltpu.VMEM_SHARED((n_nodes, d), jnp.float32),
            [pltpu.SemaphoreType.DMA for _ in range(_NBUF)],
        ],
        compiler_params=pltpu.CompilerParams(use_tc_tiling_on_sc=False),
    )
    def k(table_h, src3_h, dst3_h, et3_h, out_h,
          srcv, gidx, dstv, rows, stage, acc, gsem):
        c = lax.axis_index("c")
        s = lax.axis_index("s")
        wid = s * nc + c

        pltpu.sync_copy(src3_h.at[wid], srcv)
        pltpu.sync_copy(et3_h.at[wid], gidx)
        pltpu.sync_copy(dst3_h.at[wid], dstv)

        def irow(j, carry):
            for kk in range(_CHUNK // 16):
                csl = pl.ds(kk * 16, 16)
                gidx[j, csl] = gidx[j, csl] * n_nodes + srcv[j, csl]
            return carry

        lax.fori_loop(0, nchunk, irow, 0)

        def zrow(i, carry):
            for jj in range(d // 16):
                stage[i, pl.ds(jj * 16, 16)] = jnp.zeros((16,), jnp.float32)
            return carry

        lax.fori_loop(0, zr, zrow, 0)
        r0 = pl.multiple_of(s * rpt, 8)
        for t in range(nz):
            pltpu.sync_copy(stage, acc.at[pl.ds(r0 + t * zr, zr)])

        @pl.when(s == 0)
        def _():
            pltpu.sync_copy(stage.at[pl.ds(0, rem)],
                            acc.at[pl.ds(ns * rpt, rem)])

        plsc.subcore_barrier()

        for b in range(_NBUF):
            pltpu.async_copy(table_h.at[gidx.at[b]], rows[b], gsem[b])

        def outer(g, carry):
            for b in range(_NBUF):
                j = g * _NBUF + b
                pltpu.make_async_copy(
                    table_h.at[gidx.at[j]], rows[b], gsem[b]).wait()
                pltpu.sync_copy(rows[b], acc.at[dstv.at[j]], add=True)

                @pl.when(g < ngroups - 1)
                def _():
                    pltpu.async_copy(
                        table_h.at[gidx.at[j + _NBUF]], rows[b], gsem[b])

            return carry

        lax.fori_loop(0, ngroups, outer, 0)
        plsc.subcore_barrier()

        for t in range(nz):
            pltpu.sync_copy(acc.at[pl.ds(r0 + t * zr, zr)], stage)
            pltpu.sync_copy(stage, out_h.at[c].at[pl.ds(r0 + t * zr, zr)])

        @pl.when(s == 0)
        def _():
            pltpu.sync_copy(acc.at[pl.ds(ns * rpt, rem)],
                            stage.at[pl.ds(0, rem)])
            pltpu.sync_copy(stage.at[pl.ds(0, rem)],
                            out_h.at[c].at[pl.ds(ns * rpt, rem)])

    return k(table, src3, dst3, et3)


def kernel(x, edge_index, edge_type, W1, W2):
    src = edge_index[0]
    dst = edge_index[1]
    n, _ = x.shape
    r, _, d_h = W1.shape
    d_out = W2.shape[2]

    info = plsc.get_sparse_core_info()
    nw = info.num_cores * info.num_subcores
    e = src.shape[0]
    shape3 = (nw, e // nw // _CHUNK, _CHUNK)
    src3 = src.reshape(shape3)
    dst3 = dst.reshape(shape3)
    et3 = edge_type.reshape(shape3)

    t1 = _rel_matmul(x, W1, block_n=2000).reshape(r * n, d_h)
    p1 = _sc_gather_scatter(t1, src3, dst3, et3, n, d_h)
    t2 = _relu_sum_matmul(p1, W2, block_n=2000).reshape(r * n, d_out)
    p2 = _sc_gather_scatter(t2, src3, dst3, et3, n, d_out)
    return _pair_sum(p2, block_n=2000)


# speedup vs baseline: 83.6922x; 1.3606x over previous
import functools

import jax
import jax.numpy as jnp
from jax import lax
from jax.experimental import pallas as pl
from jax.experimental.pallas import tpu as pltpu
from jax.experimental.pallas import tpu_sc as plsc


def _mm_body(xa_ref, xb_ref, w_ref, o_ref):
    ha = jnp.dot(xa_ref[...], w_ref[0], preferred_element_type=jnp.float32)
    hb = jnp.dot(xb_ref[...], w_ref[0], preferred_element_type=jnp.float32)
    o_ref[0] = jnp.concatenate([ha, hb], axis=-1)


def _rel_matmul(x, w, block_n):
    n, d_in = x.shape
    r, _, d_out = w.shape
    nb = n // 2 // block_n
    return pl.pallas_call(
        _mm_body,
        grid=(nb, r),
        in_specs=[
            pl.BlockSpec((block_n, d_in), lambda i, ri: (i, 0)),
            pl.BlockSpec((block_n, d_in), lambda i, ri, nb=nb: (i + nb, 0)),
            pl.BlockSpec((1, d_in, d_out), lambda i, ri: (ri, 0, 0)),
        ],
        out_specs=pl.BlockSpec((1, block_n, 128), lambda i, ri: (ri, i, 0)),
        out_shape=jax.ShapeDtypeStruct((r, n // 2, 128), jnp.float32),
    )(x, x, w)


def _relu_mm_body(p_ref, w_ref, o_ref):
    half = p_ref.shape[0] // 2
    h = jnp.maximum(p_ref[:half] + p_ref[half:], 0.0)
    o = jnp.dot(h, w_ref[0], preferred_element_type=jnp.float32)
    q = half // 4
    o_ref[0] = jnp.concatenate([o[0:q], o[q:2 * q], o[2 * q:3 * q],
                                o[3 * q:4 * q]], axis=-1)


def _relu_sum_matmul(p, wp):
    r = wp.shape[0]
    rows = p.shape[0]
    half = rows // 2
    return pl.pallas_call(
        _relu_mm_body,
        grid=(r,),
        in_specs=[
            pl.BlockSpec((rows, 128), lambda ri: (0, 0)),
            pl.BlockSpec((1, 128, 32), lambda ri: (ri, 0, 0)),
        ],
        out_specs=pl.BlockSpec((1, half // 4, 128), lambda ri: (ri, 0, 0)),
        out_shape=jax.ShapeDtypeStruct((r, half // 4, 128), jnp.float32),
    )(p, wp)


def _pair_sum_body(p_ref, o_ref):
    half = p_ref.shape[0] // 2
    o_ref[...] = p_ref[:half] + p_ref[half:]


def _pair_sum(p):
    rows = p.shape[0]
    return pl.pallas_call(
        _pair_sum_body,
        grid=(1,),
        in_specs=[pl.BlockSpec((rows, 128), lambda i: (0, 0))],
        out_specs=pl.BlockSpec((rows // 2, 128), lambda i: (0, 0)),
        out_shape=jax.ShapeDtypeStruct((rows // 2, 128), jnp.float32),
    )(p)


_CHUNK = 80
_NBUF = 5


def _sc_gather_scatter(table, src3, dst3, et3, n_nodes, d, gidx_fn):
    info = plsc.get_sparse_core_info()
    nc, ns = info.num_cores, info.num_subcores
    nw = nc * ns
    nchunk = src3.shape[1]
    epw = nchunk * _CHUNK
    assert nchunk % _NBUF == 0
    ngroups = nchunk // _NBUF
    rpt = (n_nodes // ns) // 8 * 8
    rem = n_nodes - rpt * ns
    assert rem % 8 == 0 and rem <= rpt
    zr = 208
    assert rpt % zr == 0
    nz = rpt // zr

    @functools.partial(
        pl.kernel,
        out_type=jax.ShapeDtypeStruct((nc, n_nodes, d), jnp.float32),
        mesh=plsc.VectorSubcoreMesh(core_axis_name="c", subcore_axis_name="s"),
        scratch_types=[
            pltpu.VMEM((nchunk, _CHUNK), jnp.int32),
            pltpu.VMEM((nchunk, _CHUNK), jnp.int32),
            pltpu.VMEM((nchunk, _CHUNK), jnp.int32),
            [pltpu.VMEM((_CHUNK, d), jnp.float32) for _ in range(_NBUF)],
            pltpu.VMEM((zr, d), jnp.float32),
            pltpu.VMEM_SHARED((n_nodes, d), jnp.float32),
            [pltpu.SemaphoreType.DMA for _ in range(_NBUF)],
        ],
        compiler_params=pltpu.CompilerParams(use_tc_tiling_on_sc=False),
    )
    def k(table_h, src3_h, dst3_h, et3_h, out_h,
          srcv, gidx, dstv, rows, stage, acc, gsem):
        c = lax.axis_index("c")
        s = lax.axis_index("s")
        wid = s * nc + c

        pltpu.sync_copy(src3_h.at[wid], srcv)
        pltpu.sync_copy(et3_h.at[wid], gidx)
        pltpu.sync_copy(dst3_h.at[wid], dstv)

        def irow(j, carry):
            for kk in range(_CHUNK // 16):
                csl = pl.ds(kk * 16, 16)
                gidx[j, csl] = gidx_fn(gidx[j, csl], srcv[j, csl])
            return carry

        lax.fori_loop(0, nchunk, irow, 0)

        def zrow(i, carry):
            for jj in range(d // 16):
                stage[i, pl.ds(jj * 16, 16)] = jnp.zeros((16,), jnp.float32)
            return carry

        lax.fori_loop(0, zr, zrow, 0)
        r0 = pl.multiple_of(s * rpt, 8)
        for t in range(nz):
            pltpu.sync_copy(stage, acc.at[pl.ds(r0 + t * zr, zr)])

        @pl.when(s == 0)
        def _():
            pltpu.sync_copy(stage.at[pl.ds(0, rem)],
                            acc.at[pl.ds(ns * rpt, rem)])

        plsc.subcore_barrier()

        for b in range(_NBUF):
            pltpu.async_copy(table_h.at[gidx.at[b]], rows[b], gsem[b])

        def outer(g, carry):
            for b in range(_NBUF):
                j = g * _NBUF + b
                pltpu.make_async_copy(
                    table_h.at[gidx.at[j]], rows[b], gsem[b]).wait()
                pltpu.sync_copy(rows[b], acc.at[dstv.at[j]], add=True)

                @pl.when(g < ngroups - 1)
                def _():
                    pltpu.async_copy(
                        table_h.at[gidx.at[j + _NBUF]], rows[b], gsem[b])

            return carry

        lax.fori_loop(0, ngroups, outer, 0)
        plsc.subcore_barrier()

        for t in range(nz):
            pltpu.sync_copy(acc.at[pl.ds(r0 + t * zr, zr)], stage)
            pltpu.sync_copy(stage, out_h.at[c].at[pl.ds(r0 + t * zr, zr)])

        @pl.when(s == 0)
        def _():
            pltpu.sync_copy(acc.at[pl.ds(ns * rpt, rem)],
                            stage.at[pl.ds(0, rem)])
            pltpu.sync_copy(stage.at[pl.ds(0, rem)],
                            out_h.at[c].at[pl.ds(ns * rpt, rem)])

    return k(table, src3, dst3, et3)


def kernel(x, edge_index, edge_type, W1, W2):
    src = edge_index[0]
    dst = edge_index[1]
    n, _ = x.shape
    r, _, d_h = W1.shape
    d_out = W2.shape[2]

    info = plsc.get_sparse_core_info()
    nw = info.num_cores * info.num_subcores
    e = src.shape[0]
    shape3 = (nw, e // nw // _CHUNK, _CHUNK)
    src3 = src.reshape(shape3)
    dst3 = dst.reshape(shape3)
    et3 = edge_type.reshape(shape3)

    half = n // 2
    quar = n // 8

    one = jnp.int32(1)
    zero = jnp.int32(0)

    def gidx1(et, src):
        hi = jnp.where(src >= half, one, zero)
        return et * n + 2 * (src - half * hi) + hi

    def gidx2(et, src):
        t = src >> 1
        par = src & 1
        k = (jnp.where(t >= quar, one, zero)
             + jnp.where(t >= 2 * quar, one, zero)
             + jnp.where(t >= 3 * quar, one, zero))
        m = t - k * quar
        return (et * quar + m) * 8 + 2 * k + par

    W2p = jnp.zeros((r, 128, 2 * d_out), jnp.float32)
    W2p = W2p.at[:, :d_h, :d_out].set(W2).at[:, d_h:, d_out:].set(W2)

    t1 = _rel_matmul(x, W1, block_n=1000)
    p1 = _sc_gather_scatter(t1.reshape(r * n, d_h), src3, dst3, et3,
                            n, d_h, gidx1)
    t2 = _relu_sum_matmul(p1.reshape(-1, 128), W2p)
    p2 = _sc_gather_scatter(t2.reshape(r * n, d_out), src3, dst3, et3,
                            n, d_out, gidx2)
    return _pair_sum(p2.reshape(-1, 128)).reshape(n, d_out)
